# Initial kernel scaffold; baseline (speedup 1.0000x reference)
import jax
import jax.numpy as jnp
from jax.experimental import pallas as pl


def kernel(e_x, r_x, prim_adj, rela_adj, W1, b1, Wm, bg, W2, b2):
    raise NotImplementedError("write your pallas kernel here")


# speedup vs baseline: 1.0200x; 1.0200x over previous
import functools

import jax
import jax.numpy as jnp
from jax.experimental import pallas as pl
from jax.experimental.pallas import tpu as pltpu


def _stage_body(adj_ref, feat_ref, h_ref, W_ref, b_ref, Wm_ref, bg_ref, out_ref):
    t = jnp.dot(adj_ref[...], feat_ref[...], preferred_element_type=jnp.float32)
    gcn = jnp.maximum(
        jnp.dot(t, W_ref[...], preferred_element_type=jnp.float32) + b_ref[...],
        0.0,
    )
    h = h_ref[...]
    gate = jax.nn.sigmoid(
        jnp.dot(h, Wm_ref[...], preferred_element_type=jnp.float32) + bg_ref[...]
    )
    out_ref[...] = gate * gcn + (1.0 - gate) * h


def _stage(adj, feat, W, b, Wm, bg, tm):
    m, k = adj.shape
    d = feat.shape[1]
    grid = (m // tm,)
    return pl.pallas_call(
        _stage_body,
        grid=grid,
        in_specs=[
            pl.BlockSpec((tm, k), lambda i: (i, 0)),
            pl.BlockSpec((k, d), lambda i: (0, 0)),
            pl.BlockSpec((tm, d), lambda i: (i, 0)),
            pl.BlockSpec((d, d), lambda i: (0, 0)),
            pl.BlockSpec((1, d), lambda i: (0, 0)),
            pl.BlockSpec((d, d), lambda i: (0, 0)),
            pl.BlockSpec((1, d), lambda i: (0, 0)),
        ],
        out_specs=pl.BlockSpec((tm, d), lambda i: (i, 0)),
        out_shape=jax.ShapeDtypeStruct((m, d), jnp.float32),
        compiler_params=pltpu.CompilerParams(
            dimension_semantics=("arbitrary",),
        ),
    )(adj, feat, feat, W, b, Wm, bg)


@functools.partial(jax.jit, static_argnames=())
def kernel(e_x, r_x, prim_adj, rela_adj, W1, b1, Wm, bg, W2, b2):
    b1r = b1.reshape(1, -1)
    b2r = b2.reshape(1, -1)
    bgr = bg.reshape(1, -1)
    x = _stage(prim_adj, e_x, W1, b1r, Wm, bgr, tm=200)
    feat2 = jnp.concatenate((x, r_x), axis=0)
    x2 = _stage(rela_adj, feat2, W2, b2r, Wm, bgr, tm=200)
    return x2


# speedup vs baseline: 1.0224x; 1.0023x over previous
import functools

import jax
import jax.numpy as jnp
from jax.experimental import pallas as pl
from jax.experimental.pallas import tpu as pltpu


def _stage_body(adj_ref, feat_ref, h_ref, W_ref, b_ref, Wm_ref, bg_ref, out_ref):
    t = jnp.dot(adj_ref[...], feat_ref[...], preferred_element_type=jnp.float32)
    gcn = jnp.maximum(
        jnp.dot(t, W_ref[...], preferred_element_type=jnp.float32) + b_ref[...],
        0.0,
    )
    h = h_ref[...]
    gate = jax.nn.sigmoid(
        jnp.dot(h, Wm_ref[...], preferred_element_type=jnp.float32) + bg_ref[...]
    )
    out_ref[...] = gate * gcn + (1.0 - gate) * h


def _stage(adj, feat, W, b, Wm, bg, tm):
    m, k = adj.shape
    d = feat.shape[1]
    grid = (m // tm,)
    return pl.pallas_call(
        _stage_body,
        grid=grid,
        in_specs=[
            pl.BlockSpec((tm, k), lambda i: (i, 0)),
            pl.BlockSpec((k, d), lambda i: (0, 0)),
            pl.BlockSpec((tm, d), lambda i: (i, 0)),
            pl.BlockSpec((d, d), lambda i: (0, 0)),
            pl.BlockSpec((1, d), lambda i: (0, 0)),
            pl.BlockSpec((d, d), lambda i: (0, 0)),
            pl.BlockSpec((1, d), lambda i: (0, 0)),
        ],
        out_specs=pl.BlockSpec((tm, d), lambda i: (i, 0)),
        out_shape=jax.ShapeDtypeStruct((m, d), jnp.float32),
        compiler_params=pltpu.CompilerParams(
            dimension_semantics=("arbitrary",),
        ),
    )(adj, feat, feat, W, b, Wm, bg)


@functools.partial(jax.jit, static_argnames=())
def kernel(e_x, r_x, prim_adj, rela_adj, W1, b1, Wm, bg, W2, b2):
    b1r = b1.reshape(1, -1)
    b2r = b2.reshape(1, -1)
    bgr = bg.reshape(1, -1)
    x = _stage(prim_adj, e_x, W1, b1r, Wm, bgr, tm=400)
    feat2 = jnp.concatenate((x, r_x), axis=0)
    x2 = _stage(rela_adj, feat2, W2, b2r, Wm, bgr, tm=400)
    return x2


# speedup vs baseline: 1.0442x; 1.0214x over previous
import functools

import jax
import jax.numpy as jnp
from jax.experimental import pallas as pl
from jax.experimental.pallas import tpu as pltpu


def _stage_body(adj_a_ref, adj_b_ref, feat_ref, h_ref, W_ref, b_ref, Wm_ref,
                bg_ref, out_ref):
    tm = adj_a_ref.shape[0]
    feat = feat_ref[...]
    t_a = jnp.dot(adj_a_ref[...], feat, preferred_element_type=jnp.float32)
    t_b = jnp.dot(adj_b_ref[...], feat, preferred_element_type=jnp.float32)
    t = jnp.concatenate((t_a, t_b), axis=0)
    gcn = jnp.maximum(
        jnp.dot(t, W_ref[...], preferred_element_type=jnp.float32) + b_ref[...],
        0.0,
    )
    h = h_ref[...]
    gate = jax.nn.sigmoid(
        jnp.dot(h, Wm_ref[...], preferred_element_type=jnp.float32) + bg_ref[...]
    )
    out_ref[...] = gate * gcn + (1.0 - gate) * h


def _stage(adj, feat, W, b, Wm, bg, tm):
    m, k = adj.shape
    d = feat.shape[1]
    grid = (m // (2 * tm),)
    return pl.pallas_call(
        _stage_body,
        grid=grid,
        in_specs=[
            pl.BlockSpec((tm, k), lambda i: (2 * i, 0)),
            pl.BlockSpec((tm, k), lambda i: (2 * i + 1, 0)),
            pl.BlockSpec((k, d), lambda i: (0, 0)),
            pl.BlockSpec((2 * tm, d), lambda i: (i, 0)),
            pl.BlockSpec((d, d), lambda i: (0, 0)),
            pl.BlockSpec((1, d), lambda i: (0, 0)),
            pl.BlockSpec((d, d), lambda i: (0, 0)),
            pl.BlockSpec((1, d), lambda i: (0, 0)),
        ],
        out_specs=pl.BlockSpec((2 * tm, d), lambda i: (i, 0)),
        out_shape=jax.ShapeDtypeStruct((m, d), jnp.float32),
        compiler_params=pltpu.CompilerParams(
            dimension_semantics=("arbitrary",),
        ),
    )(adj, adj, feat, feat, W, b, Wm, bg)


@functools.partial(jax.jit, static_argnames=())
def kernel(e_x, r_x, prim_adj, rela_adj, W1, b1, Wm, bg, W2, b2):
    b1r = b1.reshape(1, -1)
    b2r = b2.reshape(1, -1)
    bgr = bg.reshape(1, -1)
    x = _stage(prim_adj, e_x, W1, b1r, Wm, bgr, tm=200)
    feat2 = jnp.concatenate((x, r_x), axis=0)
    x2 = _stage(rela_adj, feat2, W2, b2r, Wm, bgr, tm=200)
    return x2
